# baseline (speedup 1.0000x reference)
import functools

import jax
import jax.numpy as jnp
from jax import lax
from jax.experimental import pallas as pl
from jax.experimental.pallas import tpu as pltpu
from jax.experimental.pallas import tpu_sc as plsc


def _stage1_body(n_nodes, x_ref, bases_ref, comp_ref, root_ref, bias_ref,
                 etype_ref, src_ref, y_ref, z_ref, gidx_ref):
    r = pl.program_id(0)
    nb = bases_ref.shape[0]
    w = comp_ref[r, 0] * bases_ref[0]
    for b in range(1, nb):
        w = w + comp_ref[r, b] * bases_ref[b]
    y_ref[0] = jnp.dot(x_ref[:], w, preferred_element_type=jnp.float32)

    @pl.when(r == 0)
    def _():
        z_ref[:] = (jnp.dot(x_ref[:], root_ref[:],
                            preferred_element_type=jnp.float32)
                    + bias_ref[0])
        gidx_ref[:] = etype_ref[:] * n_nodes + src_ref[:]


def _stage3_body(agg_ref, deg_ref, z_ref, out_ref):
    a = agg_ref[0] + agg_ref[1]
    nw = deg_ref.shape[0]
    d = lax.dot_general(deg_ref[:], jnp.ones((nw, 1), jnp.float32),
                        (((0,), (0,)), ((), ())),
                        preferred_element_type=jnp.float32)
    out_ref[:] = a / jnp.maximum(d, 1.0) + z_ref[:]


def _make_sc_kernel(n_nodes, d_model, n_edges):
    info = plsc.get_sparse_core_info()
    nc, ns, lanes = info.num_cores, info.num_subcores, info.num_lanes
    nw = nc * ns
    epw = n_edges // nw
    assert epw * nw == n_edges
    chunk = 80
    nchunk = epw // chunk
    assert nchunk * chunk == epw
    nrow_blocks = n_nodes // chunk
    assert nrow_blocks * chunk == n_nodes

    mesh = plsc.VectorSubcoreMesh(core_axis_name="c", subcore_axis_name="s")

    @functools.partial(
        pl.kernel,
        out_type=(
            jax.ShapeDtypeStruct((nc, n_nodes, d_model), jnp.float32),
            jax.ShapeDtypeStruct((nw, n_nodes), jnp.float32),
        ),
        mesh=mesh,
        compiler_params=pltpu.CompilerParams(needs_layout_passes=False,
                                             use_tc_tiling_on_sc=False),
        scratch_types=[
            pltpu.VMEM((nchunk, chunk), jnp.int32),
            pltpu.VMEM((nchunk, chunk), jnp.int32),
            pltpu.VMEM((chunk, d_model), jnp.float32),
            pltpu.VMEM((chunk, d_model), jnp.float32),
            pltpu.VMEM((n_nodes,), jnp.float32),
            pltpu.VMEM_SHARED((n_nodes, d_model), jnp.float32),
            pltpu.SemaphoreType.DMA,
            pltpu.SemaphoreType.DMA,
            pltpu.SemaphoreType.DMA,
            pltpu.SemaphoreType.DMA,
            pltpu.SemaphoreType.DMA,
            pltpu.SemaphoreType.DMA,
        ],
    )
    def sc_kernel(y_hbm, gidx_hbm, dst_hbm, agg_out, deg_out,
                  gblk, dblk, rows0, rows1, deg_t, agg_s,
                  semi, semz, semg0, semg1,
                  sems0, sems1):
        cid = lax.axis_index("c")
        sid = lax.axis_index("s")
        wid = sid * nc + cid

        zero = jnp.zeros((lanes,), jnp.float32)
        one = jnp.ones((lanes,), jnp.float32)
        ngrp = chunk // lanes
        nbuf = 2
        rows = (rows0, rows1)
        semg = (semg0, semg1)
        sems = (sems0, sems1)
        nzb = (nrow_blocks + ns - 1) // ns

        pltpu.async_copy(gidx_hbm.at[pl.ds(wid * nchunk, nchunk)], gblk, semi)
        pltpu.async_copy(dst_hbm.at[pl.ds(wid * nchunk, nchunk)], dblk, semi)

        def init_body(i, _):
            r = i // (d_model // lanes)
            c = i % (d_model // lanes)
            rows0[r, pl.ds(c * lanes, lanes)] = zero
            return 0

        lax.fori_loop(0, chunk * (d_model // lanes), init_body, 0)

        def deg_zero(i, _):
            deg_t[pl.ds(i * lanes, lanes)] = zero
            return 0

        lax.fori_loop(0, n_nodes // lanes, deg_zero, 0)

        def zero_blocks(k, _):
            blk = k * ns + sid

            @pl.when(blk < nrow_blocks)
            def _():
                pltpu.async_copy(rows0, agg_s.at[pl.ds(blk * chunk, chunk)],
                                 semz)
            return 0

        def zero_drain(k, _):
            blk = k * ns + sid

            @pl.when(blk < nrow_blocks)
            def _():
                pltpu.make_async_copy(
                    rows0, agg_s.at[pl.ds(blk * chunk, chunk)], semz).wait()
            return 0

        lax.fori_loop(0, nzb, zero_blocks, 0)
        lax.fori_loop(0, nzb, zero_drain, 0)
        pltpu.make_async_copy(gidx_hbm.at[pl.ds(0, nchunk)], gblk, semi).wait()
        pltpu.make_async_copy(dst_hbm.at[pl.ds(0, nchunk)], dblk, semi).wait()
        plsc.subcore_barrier()

        def deg_update(c):
            for j in range(ngrp):
                idx = dblk[c, pl.ds(j * lanes, lanes)]
                plsc.addupdate_scatter(deg_t, [idx], one)

        for b in range(nbuf):
            pltpu.async_copy(y_hbm.at[gblk.at[b]], rows[b], semg[b])

        def edge_quad(i, _):
            for b in range(nbuf):
                c = nbuf * i + b
                pltpu.make_async_copy(y_hbm.at[gblk.at[c]], rows[b],
                                      semg[b]).wait()
                pltpu.async_copy(rows[b], agg_s.at[dblk.at[c]],
                                 sems[b], add=True)
                deg_update(c)
                pltpu.make_async_copy(rows[b], agg_s.at[dblk.at[c]],
                                      sems[b]).wait()

                @pl.when(c + nbuf < nchunk)
                def _():
                    pltpu.async_copy(y_hbm.at[gblk.at[c + nbuf]], rows[b],
                                     semg[b])
            return 0

        lax.fori_loop(0, nchunk // nbuf, edge_quad, 0)
        for c_last in range((nchunk // nbuf) * nbuf, nchunk):
            b = c_last % nbuf
            pltpu.make_async_copy(y_hbm.at[gblk.at[c_last]], rows[b],
                                  semg[b]).wait()
            pltpu.sync_copy(rows[b], agg_s.at[dblk.at[c_last]], add=True)
            deg_update(c_last)

        pltpu.async_copy(deg_t, deg_out.at[wid], semi)
        plsc.subcore_barrier()

        def out_blocks(k, _):
            blk = k * ns + sid

            @pl.when(blk < nrow_blocks)
            def _():
                r0 = blk * chunk
                pltpu.async_copy(agg_s.at[pl.ds(r0, chunk)],
                                 agg_out.at[cid, pl.ds(r0, chunk)], semz)
            return 0

        def out_drain(k, _):
            blk = k * ns + sid

            @pl.when(blk < nrow_blocks)
            def _():
                r0 = blk * chunk
                pltpu.make_async_copy(agg_s.at[pl.ds(r0, chunk)],
                                      agg_out.at[cid, pl.ds(r0, chunk)],
                                      semz).wait()
            return 0

        lax.fori_loop(0, nzb, out_blocks, 0)
        lax.fori_loop(0, nzb, out_drain, 0)
        pltpu.make_async_copy(deg_t, deg_out.at[wid], semi).wait()

    return sc_kernel


def kernel(edge_index, edge_type, embeddings, bases, comp, root, bias):
    n, d = embeddings.shape
    r_rel, b_bases = comp.shape
    e = edge_type.shape[0]

    src = edge_index[0].astype(jnp.int32)
    dst = edge_index[1].astype(jnp.int32)
    etype = edge_type.astype(jnp.int32)

    y, z, gidx = pl.pallas_call(
        functools.partial(_stage1_body, n),
        grid=(r_rel,),
        in_specs=[
            pl.BlockSpec((n, d), lambda r: (0, 0)),
            pl.BlockSpec((b_bases, d, d), lambda r: (0, 0, 0)),
            pl.BlockSpec(memory_space=pltpu.SMEM),
            pl.BlockSpec((d, d), lambda r: (0, 0)),
            pl.BlockSpec((1, d), lambda r: (0, 0)),
            pl.BlockSpec((e,), lambda r: (0,)),
            pl.BlockSpec((e,), lambda r: (0,)),
        ],
        out_specs=[
            pl.BlockSpec((1, n, d), lambda r: (r, 0, 0)),
            pl.BlockSpec((n, d), lambda r: (0, 0)),
            pl.BlockSpec((e,), lambda r: (0,)),
        ],
        out_shape=[
            jax.ShapeDtypeStruct((r_rel, n, d), jnp.float32),
            jax.ShapeDtypeStruct((n, d), jnp.float32),
            jax.ShapeDtypeStruct((e,), jnp.int32),
        ],
    )(embeddings, bases, comp, root, bias.reshape(1, d), etype, src)

    y_flat = y.reshape(r_rel * n, d)

    chunk = 80
    agg_p, deg_p = _make_sc_kernel(n, d, e)(
        y_flat, gidx.reshape(e // chunk, chunk), dst.reshape(e // chunk, chunk))

    out = pl.pallas_call(
        _stage3_body,
        in_specs=[
            pl.BlockSpec(agg_p.shape, lambda: (0, 0, 0)),
            pl.BlockSpec(deg_p.shape, lambda: (0, 0)),
            pl.BlockSpec((n, d), lambda: (0, 0)),
        ],
        out_specs=pl.BlockSpec((n, d), lambda: (0, 0)),
        out_shape=jax.ShapeDtypeStruct((n, d), jnp.float32),
    )(agg_p, deg_p, z)
    return out


# speedup vs baseline: 1.1770x; 1.1770x over previous
import functools

import jax
import jax.numpy as jnp
from jax import lax
from jax.experimental import pallas as pl
from jax.experimental.pallas import tpu as pltpu
from jax.experimental.pallas import tpu_sc as plsc


def _stage1_body(n_nodes, x_ref, bases_ref, comp_ref, root_ref, bias_ref,
                 etype_ref, src_ref, y_ref, z_ref, gidx_ref):
    r = pl.program_id(0)
    nb = bases_ref.shape[0]
    w = comp_ref[r, 0] * bases_ref[0]
    for b in range(1, nb):
        w = w + comp_ref[r, b] * bases_ref[b]
    y_ref[0] = jnp.dot(x_ref[:], w, preferred_element_type=jnp.float32)

    @pl.when(r == 0)
    def _():
        z_ref[:] = (jnp.dot(x_ref[:], root_ref[:],
                            preferred_element_type=jnp.float32)
                    + bias_ref[0])
        gidx_ref[:] = etype_ref[:] * n_nodes + src_ref[:]


def _stage3_body(agg_ref, deg_ref, z_ref, out_ref):
    a = agg_ref[0] + agg_ref[1]
    nw = deg_ref.shape[0]
    d = lax.dot_general(deg_ref[:], jnp.ones((nw, 1), jnp.float32),
                        (((0,), (0,)), ((), ())),
                        preferred_element_type=jnp.float32)
    out_ref[:] = a / jnp.maximum(d, 1.0) + z_ref[:]


def _make_sc_kernel(n_nodes, d_model, n_edges):
    info = plsc.get_sparse_core_info()
    nc, ns, lanes = info.num_cores, info.num_subcores, info.num_lanes
    nw = nc * ns
    epw = n_edges // nw
    assert epw * nw == n_edges
    chunk = 80
    nchunk = epw // chunk
    assert nchunk * chunk == epw
    nrow_blocks = n_nodes // chunk
    assert nrow_blocks * chunk == n_nodes

    mesh = plsc.VectorSubcoreMesh(core_axis_name="c", subcore_axis_name="s")

    @functools.partial(
        pl.kernel,
        out_type=jax.ShapeDtypeStruct((nw, n_nodes), jnp.float32),
        mesh=mesh,
        compiler_params=pltpu.CompilerParams(needs_layout_passes=False,
                                             use_tc_tiling_on_sc=False),
        scratch_types=[
            pltpu.VMEM((nchunk, chunk), jnp.int32),
            pltpu.VMEM((n_nodes,), jnp.float32),
            pltpu.SemaphoreType.DMA,
        ],
    )
    def deg_kernel(dst_hbm, deg_out, dblk, deg_t, semi):
        cid = lax.axis_index("c")
        sid = lax.axis_index("s")
        wid = sid * nc + cid
        zero = jnp.zeros((lanes,), jnp.float32)
        one = jnp.ones((lanes,), jnp.float32)

        pltpu.async_copy(dst_hbm.at[pl.ds(wid * nchunk, nchunk)], dblk, semi)

        def deg_zero(i, _):
            deg_t[pl.ds(i * lanes, lanes)] = zero
            return 0

        lax.fori_loop(0, n_nodes // lanes, deg_zero, 0)
        pltpu.make_async_copy(dst_hbm.at[pl.ds(0, nchunk)], dblk, semi).wait()

        ngrp_row = chunk // lanes

        def deg_count(g, _):
            c = g // ngrp_row
            j = g % ngrp_row
            idx = dblk[c, pl.ds(j * lanes, lanes)]
            plsc.addupdate_scatter(deg_t, [idx], one)
            return 0

        lax.fori_loop(0, nchunk * ngrp_row, deg_count, 0)
        pltpu.sync_copy(deg_t, deg_out.at[wid])

    @functools.partial(
        pl.kernel,
        out_type=jax.ShapeDtypeStruct((nc, n_nodes, d_model), jnp.float32),
        mesh=mesh,
        compiler_params=pltpu.CompilerParams(needs_layout_passes=False,
                                             use_tc_tiling_on_sc=False),
        scratch_types=[
            pltpu.VMEM((nchunk, chunk), jnp.int32),
            pltpu.VMEM((nchunk, chunk), jnp.int32),
            pltpu.VMEM((chunk, d_model), jnp.float32),
            pltpu.VMEM((chunk, d_model), jnp.float32),
            pltpu.VMEM((chunk, d_model), jnp.float32),
            pltpu.VMEM_SHARED((n_nodes, d_model), jnp.float32),
            pltpu.SemaphoreType.DMA,
            pltpu.SemaphoreType.DMA,
            pltpu.SemaphoreType.DMA,
            pltpu.SemaphoreType.DMA,
            pltpu.SemaphoreType.DMA,
            pltpu.SemaphoreType.DMA,
            pltpu.SemaphoreType.DMA,
            pltpu.SemaphoreType.DMA,
        ],
    )
    def sc_kernel(y_hbm, gidx_hbm, dst_hbm, agg_out,
                  gblk, dblk, rows0, rows1, rows2, agg_s,
                  semi, semz, semg0, semg1, semg2,
                  sems0, sems1, sems2):
        cid = lax.axis_index("c")
        sid = lax.axis_index("s")
        wid = sid * nc + cid

        zero = jnp.zeros((lanes,), jnp.float32)
        nbuf = 3
        rows = (rows0, rows1, rows2)
        semg = (semg0, semg1, semg2)
        sems = (sems0, sems1, sems2)
        nzb = (nrow_blocks + ns - 1) // ns

        pltpu.async_copy(gidx_hbm.at[pl.ds(wid * nchunk, nchunk)], gblk, semi)
        pltpu.async_copy(dst_hbm.at[pl.ds(wid * nchunk, nchunk)], dblk, semi)

        def init_body(i, _):
            r = i // (d_model // lanes)
            c = i % (d_model // lanes)
            rows0[r, pl.ds(c * lanes, lanes)] = zero
            return 0

        lax.fori_loop(0, chunk * (d_model // lanes), init_body, 0)

        def zero_blocks(k, _):
            blk = k * ns + sid

            @pl.when(blk < nrow_blocks)
            def _():
                pltpu.async_copy(rows0, agg_s.at[pl.ds(blk * chunk, chunk)],
                                 semz)
            return 0

        def zero_drain(k, _):
            blk = k * ns + sid

            @pl.when(blk < nrow_blocks)
            def _():
                pltpu.make_async_copy(
                    rows0, agg_s.at[pl.ds(blk * chunk, chunk)], semz).wait()
            return 0

        lax.fori_loop(0, nzb, zero_blocks, 0)
        lax.fori_loop(0, nzb, zero_drain, 0)
        pltpu.make_async_copy(gidx_hbm.at[pl.ds(0, nchunk)], gblk, semi).wait()
        pltpu.make_async_copy(dst_hbm.at[pl.ds(0, nchunk)], dblk, semi).wait()
        plsc.subcore_barrier()

        for b in range(nbuf):
            pltpu.async_copy(y_hbm.at[gblk.at[b]], rows[b], semg[b])

        def edge_quad(i, _):
            for b in range(nbuf):
                c = nbuf * i + b
                pltpu.make_async_copy(y_hbm.at[gblk.at[c]], rows[b],
                                      semg[b]).wait()
                pltpu.async_copy(rows[b], agg_s.at[dblk.at[c]],
                                 sems[b], add=True)
                pltpu.make_async_copy(rows[b], agg_s.at[dblk.at[c]],
                                      sems[b]).wait()

                @pl.when(c + nbuf < nchunk)
                def _():
                    pltpu.async_copy(y_hbm.at[gblk.at[c + nbuf]], rows[b],
                                     semg[b])
            return 0

        lax.fori_loop(0, nchunk // nbuf, edge_quad, 0)
        for c_last in range((nchunk // nbuf) * nbuf, nchunk):
            b = c_last % nbuf
            pltpu.make_async_copy(y_hbm.at[gblk.at[c_last]], rows[b],
                                  semg[b]).wait()
            pltpu.sync_copy(rows[b], agg_s.at[dblk.at[c_last]], add=True)

        plsc.subcore_barrier()

        def out_blocks(k, _):
            blk = k * ns + sid

            @pl.when(blk < nrow_blocks)
            def _():
                r0 = blk * chunk
                pltpu.async_copy(agg_s.at[pl.ds(r0, chunk)],
                                 agg_out.at[cid, pl.ds(r0, chunk)], semz)
            return 0

        def out_drain(k, _):
            blk = k * ns + sid

            @pl.when(blk < nrow_blocks)
            def _():
                r0 = blk * chunk
                pltpu.make_async_copy(agg_s.at[pl.ds(r0, chunk)],
                                      agg_out.at[cid, pl.ds(r0, chunk)],
                                      semz).wait()
            return 0

        lax.fori_loop(0, nzb, out_blocks, 0)
        lax.fori_loop(0, nzb, out_drain, 0)

    return deg_kernel, sc_kernel


def kernel(edge_index, edge_type, embeddings, bases, comp, root, bias):
    n, d = embeddings.shape
    r_rel, b_bases = comp.shape
    e = edge_type.shape[0]

    src = edge_index[0].astype(jnp.int32)
    dst = edge_index[1].astype(jnp.int32)
    etype = edge_type.astype(jnp.int32)

    y, z, gidx = pl.pallas_call(
        functools.partial(_stage1_body, n),
        grid=(r_rel,),
        in_specs=[
            pl.BlockSpec((n, d), lambda r: (0, 0)),
            pl.BlockSpec((b_bases, d, d), lambda r: (0, 0, 0)),
            pl.BlockSpec(memory_space=pltpu.SMEM),
            pl.BlockSpec((d, d), lambda r: (0, 0)),
            pl.BlockSpec((1, d), lambda r: (0, 0)),
            pl.BlockSpec((e,), lambda r: (0,)),
            pl.BlockSpec((e,), lambda r: (0,)),
        ],
        out_specs=[
            pl.BlockSpec((1, n, d), lambda r: (r, 0, 0)),
            pl.BlockSpec((n, d), lambda r: (0, 0)),
            pl.BlockSpec((e,), lambda r: (0,)),
        ],
        out_shape=[
            jax.ShapeDtypeStruct((r_rel, n, d), jnp.float32),
            jax.ShapeDtypeStruct((n, d), jnp.float32),
            jax.ShapeDtypeStruct((e,), jnp.int32),
        ],
    )(embeddings, bases, comp, root, bias.reshape(1, d), etype, src)

    y_flat = y.reshape(r_rel * n, d)

    chunk = 80
    deg_kernel, sc_kernel = _make_sc_kernel(n, d, e)
    dst2 = dst.reshape(e // chunk, chunk)
    deg_p = deg_kernel(dst2)
    agg_p = sc_kernel(y_flat, gidx.reshape(e // chunk, chunk), dst2)

    out = pl.pallas_call(
        _stage3_body,
        in_specs=[
            pl.BlockSpec(agg_p.shape, lambda: (0, 0, 0)),
            pl.BlockSpec(deg_p.shape, lambda: (0, 0)),
            pl.BlockSpec((n, d), lambda: (0, 0)),
        ],
        out_specs=pl.BlockSpec((n, d), lambda: (0, 0)),
        out_shape=jax.ShapeDtypeStruct((n, d), jnp.float32),
    )(agg_p, deg_p, z)
    return out


# speedup vs baseline: 1.1798x; 1.0024x over previous
import functools

import jax
import jax.numpy as jnp
from jax import lax
from jax.experimental import pallas as pl
from jax.experimental.pallas import tpu as pltpu
from jax.experimental.pallas import tpu_sc as plsc


def _stage1_body(x_ref, bases_ref, comp_ref, root_ref, bias_ref,
                 y_ref, z_ref):
    r = pl.program_id(0)
    nb = bases_ref.shape[0]
    w = comp_ref[r, 0] * bases_ref[0]
    for b in range(1, nb):
        w = w + comp_ref[r, b] * bases_ref[b]
    y_ref[0] = jnp.dot(x_ref[:], w, preferred_element_type=jnp.float32)

    @pl.when(r == 0)
    def _():
        z_ref[:] = (jnp.dot(x_ref[:], root_ref[:],
                            preferred_element_type=jnp.float32)
                    + bias_ref[0])


def _stage3_body(agg_ref, deg_ref, z_ref, out_ref):
    a = agg_ref[0] + agg_ref[1]
    nw = deg_ref.shape[0]
    d = lax.dot_general(deg_ref[:], jnp.ones((nw, 1), jnp.float32),
                        (((0,), (0,)), ((), ())),
                        preferred_element_type=jnp.float32)
    out_ref[:] = a / jnp.maximum(d, 1.0) + z_ref[:]


def _make_sc_kernel(n_nodes, d_model, n_edges):
    info = plsc.get_sparse_core_info()
    nc, ns, lanes = info.num_cores, info.num_subcores, info.num_lanes
    nw = nc * ns
    epw = n_edges // nw
    assert epw * nw == n_edges
    chunk = 80
    nchunk = epw // chunk
    assert nchunk * chunk == epw
    nrow_blocks = n_nodes // chunk
    assert nrow_blocks * chunk == n_nodes

    mesh = plsc.VectorSubcoreMesh(core_axis_name="c", subcore_axis_name="s")

    @functools.partial(
        pl.kernel,
        out_type=(
            jax.ShapeDtypeStruct((nw * nchunk, chunk), jnp.int32),
            jax.ShapeDtypeStruct((nw * nchunk, chunk), jnp.int32),
            jax.ShapeDtypeStruct((nw, n_nodes), jnp.float32),
        ),
        mesh=mesh,
        compiler_params=pltpu.CompilerParams(needs_layout_passes=False,
                                             use_tc_tiling_on_sc=False),
        scratch_types=[
            pltpu.VMEM((epw,), jnp.int32),
            pltpu.VMEM((epw,), jnp.int32),
            pltpu.VMEM((epw,), jnp.int32),
            pltpu.VMEM((nchunk, chunk), jnp.int32),
            pltpu.VMEM((nchunk, chunk), jnp.int32),
            pltpu.VMEM((n_nodes,), jnp.float32),
            pltpu.SemaphoreType.DMA,
        ],
    )
    def prep_kernel(src_hbm, et_hbm, dst_hbm, gidx2_out, dst2_out, deg_out,
                    srcb, etb, dstb, g2, d2, deg_t, semi):
        cid = lax.axis_index("c")
        sid = lax.axis_index("s")
        wid = sid * nc + cid
        zero = jnp.zeros((lanes,), jnp.float32)
        one = jnp.ones((lanes,), jnp.float32)
        ngrp_row = chunk // lanes

        pltpu.async_copy(src_hbm.at[pl.ds(wid * epw, epw)], srcb, semi)
        pltpu.async_copy(et_hbm.at[pl.ds(wid * epw, epw)], etb, semi)
        pltpu.async_copy(dst_hbm.at[pl.ds(wid * epw, epw)], dstb, semi)

        def deg_zero(i, _):
            deg_t[pl.ds(i * lanes, lanes)] = zero
            return 0

        lax.fori_loop(0, n_nodes // lanes, deg_zero, 0)
        pltpu.make_async_copy(src_hbm.at[pl.ds(0, epw)], srcb, semi).wait()
        pltpu.make_async_copy(et_hbm.at[pl.ds(0, epw)], etb, semi).wait()
        pltpu.make_async_copy(dst_hbm.at[pl.ds(0, epw)], dstb, semi).wait()

        def prep(g, _):
            c = g // ngrp_row
            j = g % ngrp_row
            col = j * lanes
            sv = srcb[pl.ds(g * lanes, lanes)]
            tv = etb[pl.ds(g * lanes, lanes)]
            dv = dstb[pl.ds(g * lanes, lanes)]
            g2[c, pl.ds(col, lanes)] = tv * n_nodes + sv
            d2[c, pl.ds(col, lanes)] = dv
            plsc.addupdate_scatter(deg_t, [dv], one)
            return 0

        lax.fori_loop(0, nchunk * ngrp_row, prep, 0)
        pltpu.async_copy(g2, gidx2_out.at[pl.ds(wid * nchunk, nchunk)], semi)
        pltpu.async_copy(d2, dst2_out.at[pl.ds(wid * nchunk, nchunk)], semi)
        pltpu.sync_copy(deg_t, deg_out.at[wid])
        pltpu.make_async_copy(
            g2, gidx2_out.at[pl.ds(0, nchunk)], semi).wait()
        pltpu.make_async_copy(
            d2, dst2_out.at[pl.ds(0, nchunk)], semi).wait()

    @functools.partial(
        pl.kernel,
        out_type=jax.ShapeDtypeStruct((nc, n_nodes, d_model), jnp.float32),
        mesh=mesh,
        compiler_params=pltpu.CompilerParams(needs_layout_passes=False,
                                             use_tc_tiling_on_sc=False),
        scratch_types=[
            pltpu.VMEM((nchunk, chunk), jnp.int32),
            pltpu.VMEM((nchunk, chunk), jnp.int32),
            pltpu.VMEM((chunk, d_model), jnp.float32),
            pltpu.VMEM((chunk, d_model), jnp.float32),
            pltpu.VMEM((chunk, d_model), jnp.float32),
            pltpu.VMEM_SHARED((n_nodes, d_model), jnp.float32),
            pltpu.SemaphoreType.DMA,
            pltpu.SemaphoreType.DMA,
            pltpu.SemaphoreType.DMA,
            pltpu.SemaphoreType.DMA,
            pltpu.SemaphoreType.DMA,
            pltpu.SemaphoreType.DMA,
            pltpu.SemaphoreType.DMA,
            pltpu.SemaphoreType.DMA,
        ],
    )
    def sc_kernel(y_hbm, gidx_hbm, dst_hbm, agg_out,
                  gblk, dblk, rows0, rows1, rows2, agg_s,
                  semi, semz, semg0, semg1, semg2,
                  sems0, sems1, sems2):
        cid = lax.axis_index("c")
        sid = lax.axis_index("s")
        wid = sid * nc + cid

        zero = jnp.zeros((lanes,), jnp.float32)
        nbuf = 3
        rows = (rows0, rows1, rows2)
        semg = (semg0, semg1, semg2)
        sems = (sems0, sems1, sems2)
        nzb = (nrow_blocks + ns - 1) // ns

        pltpu.async_copy(gidx_hbm.at[pl.ds(wid * nchunk, nchunk)], gblk, semi)
        pltpu.async_copy(dst_hbm.at[pl.ds(wid * nchunk, nchunk)], dblk, semi)

        def init_body(i, _):
            r = i // (d_model // lanes)
            c = i % (d_model // lanes)
            rows0[r, pl.ds(c * lanes, lanes)] = zero
            return 0

        lax.fori_loop(0, chunk * (d_model // lanes), init_body, 0)

        def zero_blocks(k, _):
            blk = k * ns + sid

            @pl.when(blk < nrow_blocks)
            def _():
                pltpu.async_copy(rows0, agg_s.at[pl.ds(blk * chunk, chunk)],
                                 semz)
            return 0

        def zero_drain(k, _):
            blk = k * ns + sid

            @pl.when(blk < nrow_blocks)
            def _():
                pltpu.make_async_copy(
                    rows0, agg_s.at[pl.ds(blk * chunk, chunk)], semz).wait()
            return 0

        lax.fori_loop(0, nzb, zero_blocks, 0)
        lax.fori_loop(0, nzb, zero_drain, 0)
        pltpu.make_async_copy(gidx_hbm.at[pl.ds(0, nchunk)], gblk, semi).wait()
        pltpu.make_async_copy(dst_hbm.at[pl.ds(0, nchunk)], dblk, semi).wait()
        plsc.subcore_barrier()

        for b in range(nbuf):
            pltpu.async_copy(y_hbm.at[gblk.at[b]], rows[b], semg[b])

        def edge_quad(i, _):
            for b in range(nbuf):
                c = nbuf * i + b
                pltpu.make_async_copy(y_hbm.at[gblk.at[c]], rows[b],
                                      semg[b]).wait()
                pltpu.async_copy(rows[b], agg_s.at[dblk.at[c]],
                                 sems[b], add=True)
                pltpu.make_async_copy(rows[b], agg_s.at[dblk.at[c]],
                                      sems[b]).wait()

                @pl.when(c + nbuf < nchunk)
                def _():
                    pltpu.async_copy(y_hbm.at[gblk.at[c + nbuf]], rows[b],
                                     semg[b])
            return 0

        lax.fori_loop(0, nchunk // nbuf, edge_quad, 0)
        for c_last in range((nchunk // nbuf) * nbuf, nchunk):
            b = c_last % nbuf
            pltpu.make_async_copy(y_hbm.at[gblk.at[c_last]], rows[b],
                                  semg[b]).wait()
            pltpu.sync_copy(rows[b], agg_s.at[dblk.at[c_last]], add=True)

        plsc.subcore_barrier()

        def out_blocks(k, _):
            blk = k * ns + sid

            @pl.when(blk < nrow_blocks)
            def _():
                r0 = blk * chunk
                pltpu.async_copy(agg_s.at[pl.ds(r0, chunk)],
                                 agg_out.at[cid, pl.ds(r0, chunk)], semz)
            return 0

        def out_drain(k, _):
            blk = k * ns + sid

            @pl.when(blk < nrow_blocks)
            def _():
                r0 = blk * chunk
                pltpu.make_async_copy(agg_s.at[pl.ds(r0, chunk)],
                                      agg_out.at[cid, pl.ds(r0, chunk)],
                                      semz).wait()
            return 0

        lax.fori_loop(0, nzb, out_blocks, 0)
        lax.fori_loop(0, nzb, out_drain, 0)

    return prep_kernel, sc_kernel


def kernel(edge_index, edge_type, embeddings, bases, comp, root, bias):
    n, d = embeddings.shape
    r_rel, b_bases = comp.shape
    e = edge_type.shape[0]

    src = edge_index[0].astype(jnp.int32)
    dst = edge_index[1].astype(jnp.int32)
    etype = edge_type.astype(jnp.int32)

    prep_kernel, sc_kernel = _make_sc_kernel(n, d, e)
    gidx2, dst2, deg_p = prep_kernel(src, etype, dst)

    y, z = pl.pallas_call(
        _stage1_body,
        grid=(r_rel,),
        in_specs=[
            pl.BlockSpec((n, d), lambda r: (0, 0)),
            pl.BlockSpec((b_bases, d, d), lambda r: (0, 0, 0)),
            pl.BlockSpec(memory_space=pltpu.SMEM),
            pl.BlockSpec((d, d), lambda r: (0, 0)),
            pl.BlockSpec((1, d), lambda r: (0, 0)),
        ],
        out_specs=[
            pl.BlockSpec((1, n, d), lambda r: (r, 0, 0)),
            pl.BlockSpec((n, d), lambda r: (0, 0)),
        ],
        out_shape=[
            jax.ShapeDtypeStruct((r_rel, n, d), jnp.float32),
            jax.ShapeDtypeStruct((n, d), jnp.float32),
        ],
    )(embeddings, bases, comp, root, bias.reshape(1, d))

    y_flat = y.reshape(r_rel * n, d)

    agg_p = sc_kernel(y_flat, gidx2, dst2)

    out = pl.pallas_call(
        _stage3_body,
        in_specs=[
            pl.BlockSpec(agg_p.shape, lambda: (0, 0, 0)),
            pl.BlockSpec(deg_p.shape, lambda: (0, 0)),
            pl.BlockSpec((n, d), lambda: (0, 0)),
        ],
        out_specs=pl.BlockSpec((n, d), lambda: (0, 0)),
        out_shape=jax.ShapeDtypeStruct((n, d), jnp.float32),
    )(agg_p, deg_p, z)
    return out


# speedup vs baseline: 1.2627x; 1.0702x over previous
import functools

import jax
import jax.numpy as jnp
from jax import lax
from jax.experimental import pallas as pl
from jax.experimental.pallas import tpu as pltpu
from jax.experimental.pallas import tpu_sc as plsc


def _stage1_body(x_ref, bases_ref, comp_ref, root_ref, bias_ref,
                 y_ref, z_ref):
    r = pl.program_id(0)
    nb = bases_ref.shape[0]
    w = comp_ref[r, 0] * bases_ref[0]
    for b in range(1, nb):
        w = w + comp_ref[r, b] * bases_ref[b]
    y_ref[0] = jnp.dot(x_ref[:], w, preferred_element_type=jnp.float32)

    @pl.when(r == 0)
    def _():
        z_ref[:] = (jnp.dot(x_ref[:], root_ref[:],
                            preferred_element_type=jnp.float32)
                    + bias_ref[0])


def _stage3_body(agg_ref, deg_ref, z_ref, out_ref):
    a = agg_ref[0] + agg_ref[1]
    nw = deg_ref.shape[0]
    d = lax.dot_general(deg_ref[:], jnp.ones((nw, 1), jnp.float32),
                        (((0,), (0,)), ((), ())),
                        preferred_element_type=jnp.float32)
    out_ref[:] = a / jnp.maximum(d, 1.0) + z_ref[:]


def _make_sc_kernel(n_nodes, d_model, n_edges):
    info = plsc.get_sparse_core_info()
    nc, ns, lanes = info.num_cores, info.num_subcores, info.num_lanes
    nw = nc * ns
    epw = n_edges // nw
    assert epw * nw == n_edges
    chunk = 80
    nchunk = epw // chunk
    assert nchunk * chunk == epw
    nrow_blocks = n_nodes // chunk
    assert nrow_blocks * chunk == n_nodes

    mesh = plsc.VectorSubcoreMesh(core_axis_name="c", subcore_axis_name="s")

    @functools.partial(
        pl.kernel,
        out_type=(
            jax.ShapeDtypeStruct((nw * nchunk, chunk), jnp.int32),
            jax.ShapeDtypeStruct((nw * nchunk, chunk), jnp.int32),
            jax.ShapeDtypeStruct((nw, n_nodes), jnp.float32),
        ),
        mesh=mesh,
        compiler_params=pltpu.CompilerParams(needs_layout_passes=False,
                                             use_tc_tiling_on_sc=False),
        scratch_types=[
            pltpu.VMEM((epw,), jnp.int32),
            pltpu.VMEM((epw,), jnp.int32),
            pltpu.VMEM((epw,), jnp.int32),
            pltpu.VMEM((nchunk, chunk), jnp.int32),
            pltpu.VMEM((nchunk, chunk), jnp.int32),
            pltpu.VMEM((n_nodes,), jnp.float32),
            pltpu.SemaphoreType.DMA,
        ],
    )
    def prep_kernel(ei_hbm, et_hbm, gidx2_out, dst2_out, deg_out,
                    srcb, etb, dstb, g2, d2, deg_t, semi):
        cid = lax.axis_index("c")
        sid = lax.axis_index("s")
        wid = sid * nc + cid
        zero = jnp.zeros((lanes,), jnp.float32)
        one = jnp.ones((lanes,), jnp.float32)
        ngrp_row = chunk // lanes

        pltpu.async_copy(ei_hbm.at[0, pl.ds(wid * epw, epw)], srcb, semi)
        pltpu.async_copy(et_hbm.at[pl.ds(wid * epw, epw)], etb, semi)
        pltpu.async_copy(ei_hbm.at[1, pl.ds(wid * epw, epw)], dstb, semi)

        def deg_zero(i, _):
            deg_t[pl.ds(i * lanes, lanes)] = zero
            return 0

        lax.fori_loop(0, n_nodes // lanes, deg_zero, 0)
        pltpu.make_async_copy(ei_hbm.at[0, pl.ds(0, epw)], srcb, semi).wait()
        pltpu.make_async_copy(et_hbm.at[pl.ds(0, epw)], etb, semi).wait()
        pltpu.make_async_copy(ei_hbm.at[1, pl.ds(0, epw)], dstb, semi).wait()

        def prep(g, _):
            c = g // ngrp_row
            j = g % ngrp_row
            col = j * lanes
            sv = srcb[pl.ds(g * lanes, lanes)]
            tv = etb[pl.ds(g * lanes, lanes)]
            dv = dstb[pl.ds(g * lanes, lanes)]
            g2[c, pl.ds(col, lanes)] = tv * n_nodes + sv
            d2[c, pl.ds(col, lanes)] = dv
            plsc.addupdate_scatter(deg_t, [dv], one)
            return 0

        lax.fori_loop(0, nchunk * ngrp_row, prep, 0)
        pltpu.async_copy(g2, gidx2_out.at[pl.ds(wid * nchunk, nchunk)], semi)
        pltpu.async_copy(d2, dst2_out.at[pl.ds(wid * nchunk, nchunk)], semi)
        pltpu.sync_copy(deg_t, deg_out.at[wid])
        pltpu.make_async_copy(
            g2, gidx2_out.at[pl.ds(0, nchunk)], semi).wait()
        pltpu.make_async_copy(
            d2, dst2_out.at[pl.ds(0, nchunk)], semi).wait()

    @functools.partial(
        pl.kernel,
        out_type=jax.ShapeDtypeStruct((nc, n_nodes, d_model), jnp.float32),
        mesh=mesh,
        compiler_params=pltpu.CompilerParams(needs_layout_passes=False,
                                             use_tc_tiling_on_sc=False),
        scratch_types=[
            pltpu.VMEM((nchunk, chunk), jnp.int32),
            pltpu.VMEM((nchunk, chunk), jnp.int32),
            pltpu.VMEM((chunk, d_model), jnp.float32),
            pltpu.VMEM((chunk, d_model), jnp.float32),
            pltpu.VMEM((chunk, d_model), jnp.float32),
            pltpu.VMEM_SHARED((n_nodes, d_model), jnp.float32),
            pltpu.SemaphoreType.DMA,
            pltpu.SemaphoreType.DMA,
            pltpu.SemaphoreType.DMA,
            pltpu.SemaphoreType.DMA,
            pltpu.SemaphoreType.DMA,
            pltpu.SemaphoreType.DMA,
            pltpu.SemaphoreType.DMA,
            pltpu.SemaphoreType.DMA,
        ],
    )
    def sc_kernel(y_hbm, gidx_hbm, dst_hbm, agg_out,
                  gblk, dblk, rows0, rows1, rows2, agg_s,
                  semi, semz, semg0, semg1, semg2,
                  sems0, sems1, sems2):
        cid = lax.axis_index("c")
        sid = lax.axis_index("s")
        wid = sid * nc + cid

        zero = jnp.zeros((lanes,), jnp.float32)
        nbuf = 3
        rows = (rows0, rows1, rows2)
        semg = (semg0, semg1, semg2)
        sems = (sems0, sems1, sems2)
        nzb = (nrow_blocks + ns - 1) // ns

        pltpu.async_copy(gidx_hbm.at[pl.ds(wid * nchunk, nchunk)], gblk, semi)
        pltpu.async_copy(dst_hbm.at[pl.ds(wid * nchunk, nchunk)], dblk, semi)

        def init_body(i, _):
            r = i // (d_model // lanes)
            c = i % (d_model // lanes)
            rows0[r, pl.ds(c * lanes, lanes)] = zero
            return 0

        lax.fori_loop(0, chunk * (d_model // lanes), init_body, 0)

        def zero_blocks(k, _):
            blk = k * ns + sid

            @pl.when(blk < nrow_blocks)
            def _():
                pltpu.async_copy(rows0, agg_s.at[pl.ds(blk * chunk, chunk)],
                                 semz)
            return 0

        def zero_drain(k, _):
            blk = k * ns + sid

            @pl.when(blk < nrow_blocks)
            def _():
                pltpu.make_async_copy(
                    rows0, agg_s.at[pl.ds(blk * chunk, chunk)], semz).wait()
            return 0

        lax.fori_loop(0, nzb, zero_blocks, 0)
        lax.fori_loop(0, nzb, zero_drain, 0)
        pltpu.make_async_copy(gidx_hbm.at[pl.ds(0, nchunk)], gblk, semi).wait()
        pltpu.make_async_copy(dst_hbm.at[pl.ds(0, nchunk)], dblk, semi).wait()
        plsc.subcore_barrier()

        for b in range(nbuf):
            pltpu.async_copy(y_hbm.at[gblk.at[b]], rows[b], semg[b])

        def edge_quad(i, _):
            for b in range(nbuf):
                c = nbuf * i + b
                pltpu.make_async_copy(y_hbm.at[gblk.at[c]], rows[b],
                                      semg[b]).wait()
                pltpu.async_copy(rows[b], agg_s.at[dblk.at[c]],
                                 sems[b], add=True)
                pltpu.make_async_copy(rows[b], agg_s.at[dblk.at[c]],
                                      sems[b]).wait()

                @pl.when(c + nbuf < nchunk)
                def _():
                    pltpu.async_copy(y_hbm.at[gblk.at[c + nbuf]], rows[b],
                                     semg[b])
            return 0

        lax.fori_loop(0, nchunk // nbuf, edge_quad, 0)
        for c_last in range((nchunk // nbuf) * nbuf, nchunk):
            b = c_last % nbuf
            pltpu.make_async_copy(y_hbm.at[gblk.at[c_last]], rows[b],
                                  semg[b]).wait()
            pltpu.sync_copy(rows[b], agg_s.at[dblk.at[c_last]], add=True)

        plsc.subcore_barrier()

        def out_blocks(k, _):
            blk = k * ns + sid

            @pl.when(blk < nrow_blocks)
            def _():
                r0 = blk * chunk
                pltpu.async_copy(agg_s.at[pl.ds(r0, chunk)],
                                 agg_out.at[cid, pl.ds(r0, chunk)], semz)
            return 0

        def out_drain(k, _):
            blk = k * ns + sid

            @pl.when(blk < nrow_blocks)
            def _():
                r0 = blk * chunk
                pltpu.make_async_copy(agg_s.at[pl.ds(r0, chunk)],
                                      agg_out.at[cid, pl.ds(r0, chunk)],
                                      semz).wait()
            return 0

        lax.fori_loop(0, nzb, out_blocks, 0)
        lax.fori_loop(0, nzb, out_drain, 0)

    return prep_kernel, sc_kernel


def kernel(edge_index, edge_type, embeddings, bases, comp, root, bias):
    n, d = embeddings.shape
    r_rel, b_bases = comp.shape
    e = edge_type.shape[0]

    prep_kernel, sc_kernel = _make_sc_kernel(n, d, e)
    gidx2, dst2, deg_p = prep_kernel(edge_index.astype(jnp.int32),
                                     edge_type.astype(jnp.int32))

    y, z = pl.pallas_call(
        _stage1_body,
        grid=(r_rel,),
        in_specs=[
            pl.BlockSpec((n, d), lambda r: (0, 0)),
            pl.BlockSpec((b_bases, d, d), lambda r: (0, 0, 0)),
            pl.BlockSpec(memory_space=pltpu.SMEM),
            pl.BlockSpec((d, d), lambda r: (0, 0)),
            pl.BlockSpec((1, d), lambda r: (0, 0)),
        ],
        out_specs=[
            pl.BlockSpec((1, n, d), lambda r: (r, 0, 0)),
            pl.BlockSpec((n, d), lambda r: (0, 0)),
        ],
        out_shape=[
            jax.ShapeDtypeStruct((r_rel, n, d), jnp.float32),
            jax.ShapeDtypeStruct((n, d), jnp.float32),
        ],
    )(embeddings, bases, comp, root, bias.reshape(1, d))

    y_flat = y.reshape(r_rel * n, d)

    agg_p = sc_kernel(y_flat, gidx2, dst2)

    out = pl.pallas_call(
        _stage3_body,
        in_specs=[
            pl.BlockSpec(agg_p.shape, lambda: (0, 0, 0)),
            pl.BlockSpec(deg_p.shape, lambda: (0, 0)),
            pl.BlockSpec((n, d), lambda: (0, 0)),
        ],
        out_specs=pl.BlockSpec((n, d), lambda: (0, 0)),
        out_shape=jax.ShapeDtypeStruct((n, d), jnp.float32),
    )(agg_p, deg_p, z)
    return out
